# Initial kernel scaffold; baseline (speedup 1.0000x reference)
import jax
import jax.numpy as jnp
from jax.experimental import pallas as pl


def kernel(drug_x, edge_index, batch, W1, b1, W2, b2):
    raise NotImplementedError("write your pallas kernel here")


# speedup vs baseline: 9.0952x; 9.0952x over previous
import functools

import jax
import jax.numpy as jnp
from jax import lax
from jax.experimental import pallas as pl
from jax.experimental.pallas import tpu as pltpu
from jax.experimental.pallas import tpu_sc as plsc

N = 10000
E = 320000
D = 128
DH = D // 2
G = 256

NC = 2
NS = 16
NW = NC * NS

NPAD = 10240
EPAD = 327680
ECH = 128
ET = EPAD // NS
CHT = ET // ECH
RS = NPAD // NS
NB = 4

_mesh = plsc.VectorSubcoreMesh(core_axis_name="c", subcore_axis_name="s")


@functools.partial(
    pl.kernel,
    out_type=jax.ShapeDtypeStruct((NC, NPAD), jnp.float32),
    mesh=_mesh,
    compiler_params=pltpu.CompilerParams(use_tc_tiling_on_sc=False),
    scratch_types=[
        pltpu.VMEM((NB, ECH), jnp.int32),
        pltpu.VMEM((ECH,), jnp.float32),
        pltpu.VMEM((RS,), jnp.float32),
        pltpu.VMEM_SHARED((NPAD,), jnp.float32),
    ],
)
def _deg_kernel(dstp_hbm, out_hbm, didx, onesb, zb, degsh):
    c = lax.axis_index("c")
    s = lax.axis_index("s")
    w = s * NC + c

    @pl.loop(0, ECH // 16)
    def _(j):
        onesb[pl.ds(j * 16, 16)] = jnp.full((16,), 1.0, jnp.float32)

    @pl.loop(0, RS // 16)
    def _(j):
        zb[pl.ds(j * 16, 16)] = jnp.zeros((16,), jnp.float32)

    pltpu.sync_copy(zb, degsh.at[pl.ds(s * RS, RS)])
    plsc.subcore_barrier()

    ebase = w * (EPAD // NW)

    @pl.loop(0, EPAD // NW // ECH, step=NB)
    def _(g0):
        for b in range(NB):
            pltpu.sync_copy(dstp_hbm.at[pl.ds(ebase + (g0 + b) * ECH, ECH)],
                            didx.at[b])
        for b in range(NB):
            pltpu.sync_copy(onesb, degsh.at[didx.at[b]], add=True)

    plsc.subcore_barrier()
    pltpu.sync_copy(degsh.at[pl.ds(s * RS, RS)], out_hbm.at[c, pl.ds(s * RS, RS)])


@functools.partial(
    pl.kernel,
    out_type=jax.ShapeDtypeStruct((NC, NPAD, DH), jnp.float32),
    mesh=_mesh,
    compiler_params=pltpu.CompilerParams(use_tc_tiling_on_sc=False),
    scratch_types=[
        pltpu.VMEM((NB, ECH), jnp.int32),
        pltpu.VMEM((NB, ECH), jnp.int32),
        pltpu.VMEM((NB, ECH, DH), jnp.float32),
        pltpu.VMEM((128, DH), jnp.float32),
        pltpu.VMEM_SHARED((NPAD, DH), jnp.float32),
        pltpu.SemaphoreType.DMA,
    ],
)
def _segsum_kernel(tab_hbm, srcp_hbm, dstp_hbm, out_hbm,
                   sidx, didx, rows, zblk, ssh, gsem):
    c = lax.axis_index("c")
    s = lax.axis_index("s")

    @pl.loop(0, 128)
    def _(i):
        @pl.loop(0, DH // 16)
        def _(j):
            zblk[i, pl.ds(j * 16, 16)] = jnp.zeros((16,), jnp.float32)

    @pl.loop(0, RS // 128)
    def _(k):
        pltpu.sync_copy(zblk, ssh.at[pl.ds(s * RS + k * 128, 128), :])

    plsc.subcore_barrier()

    ebase = s * ET
    coff = c * NPAD

    @pl.loop(0, CHT, step=NB)
    def _(g0):
        cps = []
        for b in range(NB):
            eoff = ebase + (g0 + b) * ECH
            pltpu.sync_copy(srcp_hbm.at[pl.ds(eoff, ECH)], sidx.at[b])
            pltpu.sync_copy(dstp_hbm.at[pl.ds(eoff, ECH)], didx.at[b])
            for j in range(ECH // 16):
                sl = pl.ds(j * 16, 16)
                sidx[b, sl] = sidx[b, sl] + coff
            cps.append(pltpu.async_copy(tab_hbm.at[sidx.at[b]],
                                        rows.at[b], gsem))
        for b in range(NB):
            cps[b].wait()
        for b in range(NB):
            pltpu.sync_copy(rows.at[b], ssh.at[didx.at[b]], add=True)

    plsc.subcore_barrier()
    pltpu.sync_copy(ssh.at[pl.ds(s * RS, RS), :],
                    out_hbm.at[c, pl.ds(s * RS, RS), :])


PCH = N // 16
GS = G // NS


@functools.partial(
    pl.kernel,
    out_type=(jax.ShapeDtypeStruct((NC, G, D), jnp.float32),
              jax.ShapeDtypeStruct((NC, G), jnp.float32)),
    mesh=_mesh,
    compiler_params=pltpu.CompilerParams(use_tc_tiling_on_sc=False),
    scratch_types=[
        pltpu.VMEM((16, DH), jnp.float32),
        pltpu.VMEM((16, DH), jnp.float32),
        pltpu.VMEM((16, DH), jnp.float32),
        pltpu.VMEM((16, DH), jnp.float32),
        pltpu.VMEM((16, D), jnp.float32),
        pltpu.VMEM((16,), jnp.float32),
        pltpu.VMEM((1, 16), jnp.int32),
        pltpu.VMEM((D,), jnp.float32),
        pltpu.VMEM((16,), jnp.float32),
        pltpu.VMEM((16, D), jnp.float32),
        pltpu.VMEM((16,), jnp.float32),
        pltpu.VMEM_SHARED((G, D), jnp.float32),
        pltpu.VMEM_SHARED((G,), jnp.float32),
    ],
)
def _pool_kernel(s2_hbm, h_hbm, dinv_hbm, batch_hbm, b2_hbm,
                 pout, cout,
                 bufsa, bufsb, bufha, bufhb, outb, dbuf, bidx, b2b, ones16,
                 z2d, z1d, psh, csh):
    c = lax.axis_index("c")
    s = lax.axis_index("s")
    w = s * NC + c

    ones16[...] = jnp.full((16,), 1.0, jnp.float32)
    z1d[...] = jnp.zeros((16,), jnp.float32)

    @pl.loop(0, 16)
    def _(i):
        @pl.loop(0, D // 16)
        def _(j):
            z2d[i, pl.ds(j * 16, 16)] = jnp.zeros((16,), jnp.float32)

    pltpu.sync_copy(b2_hbm, b2b)
    pltpu.sync_copy(z2d, psh.at[pl.ds(s * GS, GS), :])
    pltpu.sync_copy(z1d, csh.at[pl.ds(s * GS, GS)])
    plsc.subcore_barrier()

    @pl.loop(w, PCH, step=NW)
    def _(ch):
        r0 = ch * 16
        pltpu.sync_copy(s2_hbm.at[0, pl.ds(r0, 16), :], bufsa)
        pltpu.sync_copy(s2_hbm.at[1, pl.ds(r0, 16), :], bufsb)
        pltpu.sync_copy(h_hbm.at[0, pl.ds(r0, 16), :], bufha)
        pltpu.sync_copy(h_hbm.at[1, pl.ds(r0, 16), :], bufhb)
        pltpu.sync_copy(dinv_hbm.at[pl.ds(r0, 16)], dbuf)
        pltpu.sync_copy(batch_hbm.at[pl.ds(r0, 16)], bidx.at[0])
        dvec = dbuf[...]
        for r in range(16):
            dscal = dvec[r]
            for j in range(DH // 16):
                sl = pl.ds(j * 16, 16)
                v = (bufsa[r, sl] + bufha[r, sl]) * dscal + b2b[sl]
                outb[r, sl] = jnp.where(v >= 0.0, v, 0.01 * v)
                sh = pl.ds(DH + j * 16, 16)
                v2 = (bufsb[r, sl] + bufhb[r, sl]) * dscal + b2b[sh]
                outb[r, sh] = jnp.where(v2 >= 0.0, v2, 0.01 * v2)
        pltpu.sync_copy(outb, psh.at[bidx.at[0]], add=True)
        pltpu.sync_copy(ones16, csh.at[bidx.at[0]], add=True)

    plsc.subcore_barrier()
    pltpu.sync_copy(psh.at[pl.ds(s * GS, GS), :], pout.at[c, pl.ds(s * GS, GS), :])
    pltpu.sync_copy(csh.at[pl.ds(s * GS, GS)], cout.at[c, pl.ds(s * GS, GS)])


BR = 1024


def _k1_body(deg0_ref, deg1_ref, mask_ref, x_ref, w_ref,
             hcat_ref, dinv_ref):
    deg = deg0_ref[...] + deg1_ref[...] + mask_ref[...]
    dinv = jnp.where(deg > 0.0, lax.rsqrt(deg), 0.0)
    h = dinv * jnp.dot(x_ref[...], w_ref[...], preferred_element_type=jnp.float32)
    hcat_ref[0, :, :] = h[:, :DH]
    hcat_ref[1, :, :] = h[:, DH:]
    dinv_ref[...] = dinv


def _k3_body(s_ref, h_ref, dinv_ref, b_ref, w_ref, o_ref):
    dinv = dinv_ref[...]
    t = jnp.concatenate([s_ref[0, :, :] + h_ref[0, :, :],
                         s_ref[1, :, :] + h_ref[1, :, :]], axis=1)
    pre = dinv * t + b_ref[...]
    g = jnp.where(pre >= 0.0, pre, 0.01 * pre)
    h = dinv * jnp.dot(g, w_ref[...], preferred_element_type=jnp.float32)
    o_ref[0, :, :] = h[:, :DH]
    o_ref[1, :, :] = h[:, DH:]


def _k6_body(p0_ref, p1_ref, c0_ref, c1_ref, out_ref):
    cnt = jnp.maximum(c0_ref[...] + c1_ref[...], 1.0)
    out_ref[...] = (p0_ref[...] + p1_ref[...]) / cnt


_col_spec = pl.BlockSpec((BR, 1), lambda i: (i, 0))
_row_spec = pl.BlockSpec((BR, D), lambda i: (i, 0))
_stk_spec = pl.BlockSpec((2, BR, DH), lambda i: (0, i, 0))
_w_spec = pl.BlockSpec((D, D), lambda i: (0, 0))
_b_spec = pl.BlockSpec((1, D), lambda i: (0, 0))

_k1_call = pl.pallas_call(
    _k1_body,
    grid=(NPAD // BR,),
    in_specs=[_col_spec, _col_spec, _col_spec, _row_spec, _w_spec],
    out_specs=[_stk_spec, _col_spec],
    out_shape=[jax.ShapeDtypeStruct((2, NPAD, DH), jnp.float32),
               jax.ShapeDtypeStruct((NPAD, 1), jnp.float32)],
)

_k3_call = pl.pallas_call(
    _k3_body,
    grid=(NPAD // BR,),
    in_specs=[_stk_spec, _stk_spec, _col_spec, _b_spec, _w_spec],
    out_specs=_stk_spec,
    out_shape=jax.ShapeDtypeStruct((2, NPAD, DH), jnp.float32),
)

_k6_call = pl.pallas_call(
    _k6_body,
    in_specs=[pl.BlockSpec((G, D), lambda: (0, 0)),
              pl.BlockSpec((G, D), lambda: (0, 0)),
              pl.BlockSpec((G, 1), lambda: (0, 0)),
              pl.BlockSpec((G, 1), lambda: (0, 0))],
    out_specs=pl.BlockSpec((G, D), lambda: (0, 0)),
    out_shape=jax.ShapeDtypeStruct((G, D), jnp.float32),
)


def kernel(drug_x, edge_index, batch, W1, b1, W2, b2):
    f32 = jnp.float32
    i32 = jnp.int32

    x_pad = jnp.pad(drug_x.astype(f32), ((0, NPAD - N), (0, 0)))
    src = edge_index[0].astype(i32)
    dst = edge_index[1].astype(i32)
    pad_idx = jnp.full((EPAD - E,), N, dtype=i32)
    srcp = jnp.concatenate([src, pad_idx])
    dstp = jnp.concatenate([dst, pad_idx])
    batch32 = batch.astype(i32)
    mask_col = jnp.concatenate([jnp.ones((N, 1), f32), jnp.zeros((NPAD - N, 1), f32)])

    degp = _deg_kernel(dstp)
    h1, dinv = _k1_call(degp[0][:, None], degp[1][:, None], mask_col,
                        x_pad, W1.astype(f32))
    s1 = _segsum_kernel(h1.reshape(2 * NPAD, DH), srcp, dstp)
    h2 = _k3_call(s1, h1, dinv, b1.astype(f32)[None, :], W2.astype(f32))
    s2 = _segsum_kernel(h2.reshape(2 * NPAD, DH), srcp, dstp)
    poolp, cntp = _pool_kernel(s2, h2, dinv[:, 0], batch32,
                               b2.astype(f32))
    out = _k6_call(poolp[0], poolp[1], cntp[0][:, None], cntp[1][:, None])
    return out


# speedup vs baseline: 11.6270x; 1.2784x over previous
import functools

import jax
import jax.numpy as jnp
from jax import lax
from jax.experimental import pallas as pl
from jax.experimental.pallas import tpu as pltpu
from jax.experimental.pallas import tpu_sc as plsc

N = 10000
E = 320000
D = 128
DH = D // 2
G = 256

NC = 2
NS = 16
NW = NC * NS

NPAD = 10240
EPAD = 327680
ECH = 128
ET = EPAD // NS
CHT = ET // ECH
RS = NPAD // NS
NB = 4

_mesh = plsc.VectorSubcoreMesh(core_axis_name="c", subcore_axis_name="s")


@functools.partial(
    pl.kernel,
    out_type=jax.ShapeDtypeStruct((NC, NPAD), jnp.float32),
    mesh=_mesh,
    compiler_params=pltpu.CompilerParams(use_tc_tiling_on_sc=False),
    scratch_types=[
        pltpu.VMEM((NB, ECH), jnp.int32),
        pltpu.VMEM((ECH,), jnp.float32),
        pltpu.VMEM((RS,), jnp.float32),
        pltpu.VMEM_SHARED((NPAD,), jnp.float32),
    ],
)
def _deg_kernel(dst2d_hbm, out_hbm, didx, onesb, zb, degsh):
    c = lax.axis_index("c")
    s = lax.axis_index("s")
    w = s * NC + c

    @pl.loop(0, ECH // 16)
    def _(j):
        onesb[pl.ds(j * 16, 16)] = jnp.full((16,), 1.0, jnp.float32)

    @pl.loop(0, RS // 16)
    def _(j):
        zb[pl.ds(j * 16, 16)] = jnp.zeros((16,), jnp.float32)

    pltpu.sync_copy(zb, degsh.at[pl.ds(s * RS, RS)])
    plsc.subcore_barrier()

    cb = w * (EPAD // NW // ECH)

    @pl.loop(0, EPAD // NW // ECH, step=NB)
    def _(g0):
        pltpu.sync_copy(dst2d_hbm.at[pl.ds(cb + g0, NB), :], didx)
        for b in range(NB):
            pltpu.sync_copy(onesb, degsh.at[didx.at[b]], add=True)

    plsc.subcore_barrier()
    pltpu.sync_copy(degsh.at[pl.ds(s * RS, RS)], out_hbm.at[c, pl.ds(s * RS, RS)])


@functools.partial(
    pl.kernel,
    out_type=jax.ShapeDtypeStruct((NC, NPAD, DH), jnp.float32),
    mesh=_mesh,
    compiler_params=pltpu.CompilerParams(use_tc_tiling_on_sc=False),
    scratch_types=[
        pltpu.VMEM((NB, ECH), jnp.int32),
        pltpu.VMEM((NB, ECH), jnp.int32),
        pltpu.VMEM((NB, ECH), jnp.int32),
        pltpu.VMEM((NB, ECH), jnp.int32),
        pltpu.VMEM((NB, ECH, DH), jnp.float32),
        pltpu.VMEM((NB, ECH, DH), jnp.float32),
        pltpu.VMEM((128, DH), jnp.float32),
        pltpu.VMEM_SHARED((NPAD, DH), jnp.float32),
        pltpu.SemaphoreType.DMA,
        pltpu.SemaphoreType.DMA,
    ],
)
def _segsum_kernel(tab_hbm, src2d_hbm, dst2d_hbm, out_hbm,
                   sidxa, didxa, sidxb, didxb, rowsa, rowsb, zblk, ssh,
                   sema, semb):
    c = lax.axis_index("c")
    s = lax.axis_index("s")

    @pl.loop(0, 128)
    def _(i):
        @pl.loop(0, DH // 16)
        def _(j):
            zblk[i, pl.ds(j * 16, 16)] = jnp.zeros((16,), jnp.float32)

    @pl.loop(0, RS // 128)
    def _(k):
        pltpu.sync_copy(zblk, ssh.at[pl.ds(s * RS + k * 128, 128), :])

    plsc.subcore_barrier()

    cb = s * CHT
    coff = c * NPAD
    NG = CHT // NB

    def fire(grp, sidx, didx, rows, sem):
        row0 = cb + grp * NB
        pltpu.sync_copy(src2d_hbm.at[pl.ds(row0, NB), :], sidx)
        pltpu.sync_copy(dst2d_hbm.at[pl.ds(row0, NB), :], didx)
        for b in range(NB):
            for j in range(ECH // 16):
                sl = pl.ds(j * 16, 16)
                sidx[b, sl] = sidx[b, sl] + coff
            pltpu.async_copy(tab_hbm.at[sidx.at[b]], rows.at[b], sem)

    def drain(sidx, didx, rows, sem):
        for b in range(NB):
            pltpu.make_async_copy(tab_hbm.at[sidx.at[b]], rows.at[b],
                                  sem).wait()
        for b in range(NB):
            pltpu.sync_copy(rows.at[b], ssh.at[didx.at[b]], add=True)

    fire(0, sidxa, didxa, rowsa, sema)

    @pl.loop(0, NG // 2)
    def _(k):
        fire(2 * k + 1, sidxb, didxb, rowsb, semb)
        drain(sidxa, didxa, rowsa, sema)

        @pl.when(k < NG // 2 - 1)
        def _():
            fire(2 * k + 2, sidxa, didxa, rowsa, sema)

        drain(sidxb, didxb, rowsb, semb)

    plsc.subcore_barrier()
    pltpu.sync_copy(ssh.at[pl.ds(s * RS, RS), :],
                    out_hbm.at[c, pl.ds(s * RS, RS), :])


PCH = N // 16
GS = G // NS


@functools.partial(
    pl.kernel,
    out_type=(jax.ShapeDtypeStruct((NC, G, D), jnp.float32),
              jax.ShapeDtypeStruct((NC, G), jnp.float32)),
    mesh=_mesh,
    compiler_params=pltpu.CompilerParams(use_tc_tiling_on_sc=False),
    scratch_types=[
        pltpu.VMEM((16, DH), jnp.float32),
        pltpu.VMEM((16, DH), jnp.float32),
        pltpu.VMEM((16, DH), jnp.float32),
        pltpu.VMEM((16, DH), jnp.float32),
        pltpu.VMEM((16, D), jnp.float32),
        pltpu.VMEM((16,), jnp.float32),
        pltpu.VMEM((1, 16), jnp.int32),
        pltpu.VMEM((D,), jnp.float32),
        pltpu.VMEM((16,), jnp.float32),
        pltpu.VMEM((16, D), jnp.float32),
        pltpu.VMEM((16,), jnp.float32),
        pltpu.VMEM_SHARED((G, D), jnp.float32),
        pltpu.VMEM_SHARED((G,), jnp.float32),
    ],
)
def _pool_kernel(s2_hbm, h_hbm, dinv_hbm, batch_hbm, b2_hbm,
                 pout, cout,
                 bufsa, bufsb, bufha, bufhb, outb, dbuf, bidx, b2b, ones16,
                 z2d, z1d, psh, csh):
    c = lax.axis_index("c")
    s = lax.axis_index("s")
    w = s * NC + c

    ones16[...] = jnp.full((16,), 1.0, jnp.float32)
    z1d[...] = jnp.zeros((16,), jnp.float32)

    @pl.loop(0, 16)
    def _(i):
        @pl.loop(0, D // 16)
        def _(j):
            z2d[i, pl.ds(j * 16, 16)] = jnp.zeros((16,), jnp.float32)

    pltpu.sync_copy(b2_hbm, b2b)
    pltpu.sync_copy(z2d, psh.at[pl.ds(s * GS, GS), :])
    pltpu.sync_copy(z1d, csh.at[pl.ds(s * GS, GS)])
    plsc.subcore_barrier()

    @pl.loop(w, PCH, step=NW)
    def _(ch):
        r0 = ch * 16
        pltpu.sync_copy(s2_hbm.at[0, pl.ds(r0, 16), :], bufsa)
        pltpu.sync_copy(s2_hbm.at[1, pl.ds(r0, 16), :], bufsb)
        pltpu.sync_copy(h_hbm.at[0, pl.ds(r0, 16), :], bufha)
        pltpu.sync_copy(h_hbm.at[1, pl.ds(r0, 16), :], bufhb)
        pltpu.sync_copy(dinv_hbm.at[pl.ds(r0, 16)], dbuf)
        pltpu.sync_copy(batch_hbm.at[pl.ds(r0, 16)], bidx.at[0])
        dvec = dbuf[...]
        for r in range(16):
            dscal = dvec[r]
            for j in range(DH // 16):
                sl = pl.ds(j * 16, 16)
                v = (bufsa[r, sl] + bufha[r, sl]) * dscal + b2b[sl]
                outb[r, sl] = jnp.where(v >= 0.0, v, 0.01 * v)
                sh = pl.ds(DH + j * 16, 16)
                v2 = (bufsb[r, sl] + bufhb[r, sl]) * dscal + b2b[sh]
                outb[r, sh] = jnp.where(v2 >= 0.0, v2, 0.01 * v2)
        pltpu.sync_copy(outb, psh.at[bidx.at[0]], add=True)
        pltpu.sync_copy(ones16, csh.at[bidx.at[0]], add=True)

    plsc.subcore_barrier()
    pltpu.sync_copy(psh.at[pl.ds(s * GS, GS), :], pout.at[c, pl.ds(s * GS, GS), :])
    pltpu.sync_copy(csh.at[pl.ds(s * GS, GS)], cout.at[c, pl.ds(s * GS, GS)])


BR = 1024


def _k1_body(deg0_ref, deg1_ref, mask_ref, x_ref, w_ref,
             hcat_ref, dinv_ref):
    deg = deg0_ref[...] + deg1_ref[...] + mask_ref[...]
    dinv = jnp.where(deg > 0.0, lax.rsqrt(deg), 0.0)
    h = dinv * jnp.dot(x_ref[...], w_ref[...], preferred_element_type=jnp.float32)
    hcat_ref[0, :, :] = h[:, :DH]
    hcat_ref[1, :, :] = h[:, DH:]
    dinv_ref[...] = dinv


def _k3_body(s_ref, h_ref, dinv_ref, b_ref, w_ref, o_ref):
    dinv = dinv_ref[...]
    t = jnp.concatenate([s_ref[0, :, :] + h_ref[0, :, :],
                         s_ref[1, :, :] + h_ref[1, :, :]], axis=1)
    pre = dinv * t + b_ref[...]
    g = jnp.where(pre >= 0.0, pre, 0.01 * pre)
    h = dinv * jnp.dot(g, w_ref[...], preferred_element_type=jnp.float32)
    o_ref[0, :, :] = h[:, :DH]
    o_ref[1, :, :] = h[:, DH:]


def _k6_body(p0_ref, p1_ref, c0_ref, c1_ref, out_ref):
    cnt = jnp.maximum(c0_ref[...] + c1_ref[...], 1.0)
    out_ref[...] = (p0_ref[...] + p1_ref[...]) / cnt


_col_spec = pl.BlockSpec((BR, 1), lambda i: (i, 0))
_row_spec = pl.BlockSpec((BR, D), lambda i: (i, 0))
_stk_spec = pl.BlockSpec((2, BR, DH), lambda i: (0, i, 0))
_w_spec = pl.BlockSpec((D, D), lambda i: (0, 0))
_b_spec = pl.BlockSpec((1, D), lambda i: (0, 0))

_k1_call = pl.pallas_call(
    _k1_body,
    grid=(NPAD // BR,),
    in_specs=[_col_spec, _col_spec, _col_spec, _row_spec, _w_spec],
    out_specs=[_stk_spec, _col_spec],
    out_shape=[jax.ShapeDtypeStruct((2, NPAD, DH), jnp.float32),
               jax.ShapeDtypeStruct((NPAD, 1), jnp.float32)],
)

_k3_call = pl.pallas_call(
    _k3_body,
    grid=(NPAD // BR,),
    in_specs=[_stk_spec, _stk_spec, _col_spec, _b_spec, _w_spec],
    out_specs=_stk_spec,
    out_shape=jax.ShapeDtypeStruct((2, NPAD, DH), jnp.float32),
)

_k6_call = pl.pallas_call(
    _k6_body,
    in_specs=[pl.BlockSpec((G, D), lambda: (0, 0)),
              pl.BlockSpec((G, D), lambda: (0, 0)),
              pl.BlockSpec((G, 1), lambda: (0, 0)),
              pl.BlockSpec((G, 1), lambda: (0, 0))],
    out_specs=pl.BlockSpec((G, D), lambda: (0, 0)),
    out_shape=jax.ShapeDtypeStruct((G, D), jnp.float32),
)


def kernel(drug_x, edge_index, batch, W1, b1, W2, b2):
    f32 = jnp.float32
    i32 = jnp.int32

    x_pad = jnp.pad(drug_x.astype(f32), ((0, NPAD - N), (0, 0)))
    src = edge_index[0].astype(i32)
    dst = edge_index[1].astype(i32)
    pad_idx = jnp.full((EPAD - E,), N, dtype=i32)
    srcp = jnp.concatenate([src, pad_idx])
    dstp = jnp.concatenate([dst, pad_idx])
    batch32 = batch.astype(i32)
    mask_col = jnp.concatenate([jnp.ones((N, 1), f32), jnp.zeros((NPAD - N, 1), f32)])

    src2d = srcp.reshape(EPAD // ECH, ECH)
    dst2d = dstp.reshape(EPAD // ECH, ECH)

    degp = _deg_kernel(dst2d)
    h1, dinv = _k1_call(degp[0][:, None], degp[1][:, None], mask_col,
                        x_pad, W1.astype(f32))
    s1 = _segsum_kernel(h1.reshape(2 * NPAD, DH), src2d, dst2d)
    h2 = _k3_call(s1, h1, dinv, b1.astype(f32)[None, :], W2.astype(f32))
    s2 = _segsum_kernel(h2.reshape(2 * NPAD, DH), src2d, dst2d)
    poolp, cntp = _pool_kernel(s2, h2, dinv[:, 0], batch32,
                               b2.astype(f32))
    out = _k6_call(poolp[0], poolp[1], cntp[0][:, None], cntp[1][:, None])
    return out


# speedup vs baseline: 11.9431x; 1.0272x over previous
import functools

import jax
import jax.numpy as jnp
from jax import lax
from jax.experimental import pallas as pl
from jax.experimental.pallas import tpu as pltpu
from jax.experimental.pallas import tpu_sc as plsc

N = 10000
E = 320000
D = 128
DH = D // 2
G = 256

NC = 2
NS = 16
NW = NC * NS

NPAD = 10240
EPAD = 327680
ECH = 128
ET = EPAD // NS
CHT = ET // ECH
RS = NPAD // NS
NB = 4

_mesh = plsc.VectorSubcoreMesh(core_axis_name="c", subcore_axis_name="s")


@functools.partial(
    pl.kernel,
    out_type=jax.ShapeDtypeStruct((NC, NPAD), jnp.float32),
    mesh=_mesh,
    compiler_params=pltpu.CompilerParams(use_tc_tiling_on_sc=False),
    scratch_types=[
        pltpu.VMEM((NB, ECH), jnp.int32),
        pltpu.VMEM((ECH,), jnp.float32),
        pltpu.VMEM((RS,), jnp.float32),
        pltpu.VMEM_SHARED((NPAD,), jnp.float32),
    ],
)
def _deg_kernel(dst2d_hbm, out_hbm, didx, onesb, zb, degsh):
    c = lax.axis_index("c")
    s = lax.axis_index("s")
    w = s * NC + c

    @pl.loop(0, ECH // 16)
    def _(j):
        onesb[pl.ds(j * 16, 16)] = jnp.full((16,), 1.0, jnp.float32)

    @pl.loop(0, RS // 16)
    def _(j):
        zb[pl.ds(j * 16, 16)] = jnp.zeros((16,), jnp.float32)

    pltpu.sync_copy(zb, degsh.at[pl.ds(s * RS, RS)])
    plsc.subcore_barrier()

    cb = w * (EPAD // NW // ECH)

    @pl.loop(0, EPAD // NW // ECH, step=NB)
    def _(g0):
        pltpu.sync_copy(dst2d_hbm.at[pl.ds(cb + g0, NB), :], didx)
        for b in range(NB):
            pltpu.sync_copy(onesb, degsh.at[didx.at[b]], add=True)

    plsc.subcore_barrier()
    pltpu.sync_copy(degsh.at[pl.ds(s * RS, RS)], out_hbm.at[c, pl.ds(s * RS, RS)])


@functools.partial(
    pl.kernel,
    out_type=jax.ShapeDtypeStruct((NC, NPAD, DH), jnp.float32),
    mesh=_mesh,
    compiler_params=pltpu.CompilerParams(use_tc_tiling_on_sc=False),
    scratch_types=[
        pltpu.VMEM((NB, ECH), jnp.int32),
        pltpu.VMEM((NB, ECH), jnp.int32),
        pltpu.VMEM((NB, ECH), jnp.int32),
        pltpu.VMEM((NB, ECH), jnp.int32),
        pltpu.VMEM((NB, ECH, DH), jnp.float32),
        pltpu.VMEM((NB, ECH, DH), jnp.float32),
        pltpu.VMEM((128, DH), jnp.float32),
        pltpu.VMEM_SHARED((NPAD, DH), jnp.float32),
        pltpu.SemaphoreType.DMA,
        pltpu.SemaphoreType.DMA,
        pltpu.SemaphoreType.DMA,
        pltpu.SemaphoreType.DMA,
    ],
)
def _segsum_kernel(tab_hbm, src3d_hbm, dst2d_hbm, out_hbm,
                   sidxa, didxa, sidxb, didxb, rowsa, rowsb, zblk, ssh,
                   sga, sgb, ssa, ssb):
    c = lax.axis_index("c")
    s = lax.axis_index("s")

    @pl.loop(0, 128)
    def _(i):
        @pl.loop(0, DH // 16)
        def _(j):
            zblk[i, pl.ds(j * 16, 16)] = jnp.zeros((16,), jnp.float32)

    @pl.loop(0, RS // 128)
    def _(k):
        pltpu.sync_copy(zblk, ssh.at[pl.ds(s * RS + k * 128, 128), :])

    plsc.subcore_barrier()

    cb = s * CHT
    NG = CHT // NB

    def fire_g(grp, sidx, didx, rows, sem):
        row0 = cb + grp * NB
        pltpu.sync_copy(src3d_hbm.at[c, pl.ds(row0, NB), :], sidx)
        pltpu.sync_copy(dst2d_hbm.at[pl.ds(row0, NB), :], didx)
        for b in range(NB):
            pltpu.async_copy(tab_hbm.at[sidx.at[b]], rows.at[b], sem)

    def wait_g(sidx, rows, sem):
        for b in range(NB):
            pltpu.make_async_copy(tab_hbm.at[sidx.at[b]], rows.at[b],
                                  sem).wait()

    def fire_s(didx, rows, sem):
        for b in range(NB):
            pltpu.async_copy(rows.at[b], ssh.at[didx.at[b]], sem, add=True)

    def wait_s(didx, rows, sem):
        for b in range(NB):
            pltpu.make_async_copy(rows.at[b], ssh.at[didx.at[b]], sem).wait()

    fire_g(0, sidxa, didxa, rowsa, sga)

    @pl.loop(0, NG // 2)
    def _(k):
        wait_g(sidxa, rowsa, sga)
        fire_s(didxa, rowsa, ssa)

        @pl.when(k > 0)
        def _():
            wait_s(didxb, rowsb, ssb)

        fire_g(2 * k + 1, sidxb, didxb, rowsb, sgb)
        wait_g(sidxb, rowsb, sgb)
        fire_s(didxb, rowsb, ssb)
        wait_s(didxa, rowsa, ssa)

        @pl.when(k < NG // 2 - 1)
        def _():
            fire_g(2 * k + 2, sidxa, didxa, rowsa, sga)

    wait_s(didxb, rowsb, ssb)

    plsc.subcore_barrier()
    pltpu.sync_copy(ssh.at[pl.ds(s * RS, RS), :],
                    out_hbm.at[c, pl.ds(s * RS, RS), :])


PCH = N // 16
GS = G // NS


@functools.partial(
    pl.kernel,
    out_type=(jax.ShapeDtypeStruct((NC, G, D), jnp.float32),
              jax.ShapeDtypeStruct((NC, G), jnp.float32)),
    mesh=_mesh,
    compiler_params=pltpu.CompilerParams(use_tc_tiling_on_sc=False),
    scratch_types=[
        pltpu.VMEM((16, DH), jnp.float32),
        pltpu.VMEM((16, DH), jnp.float32),
        pltpu.VMEM((16, DH), jnp.float32),
        pltpu.VMEM((16, DH), jnp.float32),
        pltpu.VMEM((16, D), jnp.float32),
        pltpu.VMEM((16,), jnp.float32),
        pltpu.VMEM((1, 16), jnp.int32),
        pltpu.VMEM((D,), jnp.float32),
        pltpu.VMEM((16,), jnp.float32),
        pltpu.VMEM((16, D), jnp.float32),
        pltpu.VMEM((16,), jnp.float32),
        pltpu.VMEM_SHARED((G, D), jnp.float32),
        pltpu.VMEM_SHARED((G,), jnp.float32),
    ],
)
def _pool_kernel(s2_hbm, h_hbm, dinv_hbm, batch_hbm, b2_hbm,
                 pout, cout,
                 bufsa, bufsb, bufha, bufhb, outb, dbuf, bidx, b2b, ones16,
                 z2d, z1d, psh, csh):
    c = lax.axis_index("c")
    s = lax.axis_index("s")
    w = s * NC + c

    ones16[...] = jnp.full((16,), 1.0, jnp.float32)
    z1d[...] = jnp.zeros((16,), jnp.float32)

    @pl.loop(0, 16)
    def _(i):
        @pl.loop(0, D // 16)
        def _(j):
            z2d[i, pl.ds(j * 16, 16)] = jnp.zeros((16,), jnp.float32)

    pltpu.sync_copy(b2_hbm, b2b)
    pltpu.sync_copy(z2d, psh.at[pl.ds(s * GS, GS), :])
    pltpu.sync_copy(z1d, csh.at[pl.ds(s * GS, GS)])
    plsc.subcore_barrier()

    @pl.loop(w, PCH, step=NW)
    def _(ch):
        r0 = ch * 16
        pltpu.sync_copy(s2_hbm.at[0, pl.ds(r0, 16), :], bufsa)
        pltpu.sync_copy(s2_hbm.at[1, pl.ds(r0, 16), :], bufsb)
        pltpu.sync_copy(h_hbm.at[0, pl.ds(r0, 16), :], bufha)
        pltpu.sync_copy(h_hbm.at[1, pl.ds(r0, 16), :], bufhb)
        pltpu.sync_copy(dinv_hbm.at[pl.ds(r0, 16)], dbuf)
        pltpu.sync_copy(batch_hbm.at[pl.ds(r0, 16)], bidx.at[0])
        dvec = dbuf[...]
        for r in range(16):
            dscal = dvec[r]
            for j in range(DH // 16):
                sl = pl.ds(j * 16, 16)
                v = (bufsa[r, sl] + bufha[r, sl]) * dscal + b2b[sl]
                outb[r, sl] = jnp.where(v >= 0.0, v, 0.01 * v)
                sh = pl.ds(DH + j * 16, 16)
                v2 = (bufsb[r, sl] + bufhb[r, sl]) * dscal + b2b[sh]
                outb[r, sh] = jnp.where(v2 >= 0.0, v2, 0.01 * v2)
        pltpu.sync_copy(outb, psh.at[bidx.at[0]], add=True)
        pltpu.sync_copy(ones16, csh.at[bidx.at[0]], add=True)

    plsc.subcore_barrier()
    pltpu.sync_copy(psh.at[pl.ds(s * GS, GS), :], pout.at[c, pl.ds(s * GS, GS), :])
    pltpu.sync_copy(csh.at[pl.ds(s * GS, GS)], cout.at[c, pl.ds(s * GS, GS)])


BR = 1024


def _k1_body(deg0_ref, deg1_ref, mask_ref, x_ref, w_ref,
             hcat_ref, dinv_ref):
    deg = deg0_ref[...] + deg1_ref[...] + mask_ref[...]
    dinv = jnp.where(deg > 0.0, lax.rsqrt(deg), 0.0)
    h = dinv * jnp.dot(x_ref[...], w_ref[...], preferred_element_type=jnp.float32)
    hcat_ref[0, :, :] = h[:, :DH]
    hcat_ref[1, :, :] = h[:, DH:]
    dinv_ref[...] = dinv


def _k3_body(s_ref, h_ref, dinv_ref, b_ref, w_ref, o_ref):
    dinv = dinv_ref[...]
    t = jnp.concatenate([s_ref[0, :, :] + h_ref[0, :, :],
                         s_ref[1, :, :] + h_ref[1, :, :]], axis=1)
    pre = dinv * t + b_ref[...]
    g = jnp.where(pre >= 0.0, pre, 0.01 * pre)
    h = dinv * jnp.dot(g, w_ref[...], preferred_element_type=jnp.float32)
    o_ref[0, :, :] = h[:, :DH]
    o_ref[1, :, :] = h[:, DH:]


def _k6_body(p0_ref, p1_ref, c0_ref, c1_ref, out_ref):
    cnt = jnp.maximum(c0_ref[...] + c1_ref[...], 1.0)
    out_ref[...] = (p0_ref[...] + p1_ref[...]) / cnt


_col_spec = pl.BlockSpec((BR, 1), lambda i: (i, 0))
_row_spec = pl.BlockSpec((BR, D), lambda i: (i, 0))
_stk_spec = pl.BlockSpec((2, BR, DH), lambda i: (0, i, 0))
_w_spec = pl.BlockSpec((D, D), lambda i: (0, 0))
_b_spec = pl.BlockSpec((1, D), lambda i: (0, 0))

_k1_call = pl.pallas_call(
    _k1_body,
    grid=(NPAD // BR,),
    in_specs=[_col_spec, _col_spec, _col_spec, _row_spec, _w_spec],
    out_specs=[_stk_spec, _col_spec],
    out_shape=[jax.ShapeDtypeStruct((2, NPAD, DH), jnp.float32),
               jax.ShapeDtypeStruct((NPAD, 1), jnp.float32)],
)

_k3_call = pl.pallas_call(
    _k3_body,
    grid=(NPAD // BR,),
    in_specs=[_stk_spec, _stk_spec, _col_spec, _b_spec, _w_spec],
    out_specs=_stk_spec,
    out_shape=jax.ShapeDtypeStruct((2, NPAD, DH), jnp.float32),
)

_k6_call = pl.pallas_call(
    _k6_body,
    in_specs=[pl.BlockSpec((G, D), lambda: (0, 0)),
              pl.BlockSpec((G, D), lambda: (0, 0)),
              pl.BlockSpec((G, 1), lambda: (0, 0)),
              pl.BlockSpec((G, 1), lambda: (0, 0))],
    out_specs=pl.BlockSpec((G, D), lambda: (0, 0)),
    out_shape=jax.ShapeDtypeStruct((G, D), jnp.float32),
)


def kernel(drug_x, edge_index, batch, W1, b1, W2, b2):
    f32 = jnp.float32
    i32 = jnp.int32

    x_pad = jnp.pad(drug_x.astype(f32), ((0, NPAD - N), (0, 0)))
    src = edge_index[0].astype(i32)
    dst = edge_index[1].astype(i32)
    pad_idx = jnp.full((EPAD - E,), N, dtype=i32)
    srcp = jnp.concatenate([src, pad_idx])
    dstp = jnp.concatenate([dst, pad_idx])
    batch32 = batch.astype(i32)
    mask_col = jnp.concatenate([jnp.ones((N, 1), f32), jnp.zeros((NPAD - N, 1), f32)])

    src2d = srcp.reshape(EPAD // ECH, ECH)
    dst2d = dstp.reshape(EPAD // ECH, ECH)
    src3d = jnp.stack([src2d, src2d + NPAD])

    degp = _deg_kernel(dst2d)
    h1, dinv = _k1_call(degp[0][:, None], degp[1][:, None], mask_col,
                        x_pad, W1.astype(f32))
    s1 = _segsum_kernel(h1.reshape(2 * NPAD, DH), src3d, dst2d)
    h2 = _k3_call(s1, h1, dinv, b1.astype(f32)[None, :], W2.astype(f32))
    s2 = _segsum_kernel(h2.reshape(2 * NPAD, DH), src3d, dst2d)
    poolp, cntp = _pool_kernel(s2, h2, dinv[:, 0], batch32,
                               b2.astype(f32))
    out = _k6_call(poolp[0], poolp[1], cntp[0][:, None], cntp[1][:, None])
    return out


# speedup vs baseline: 11.9581x; 1.0013x over previous
import functools

import jax
import jax.numpy as jnp
from jax import lax
from jax.experimental import pallas as pl
from jax.experimental.pallas import tpu as pltpu
from jax.experimental.pallas import tpu_sc as plsc

N = 10000
E = 320000
D = 128
DH = D // 2
G = 256

NC = 2
NS = 16
NW = NC * NS

NPAD = 10240
EPAD = 327680
ECH = 128
ET = EPAD // NS
CHT = ET // ECH
RS = NPAD // NS
NB = 4
SB = 4

_mesh = plsc.VectorSubcoreMesh(core_axis_name="c", subcore_axis_name="s")


@functools.partial(
    pl.kernel,
    out_type=jax.ShapeDtypeStruct((NC, NPAD), jnp.float32),
    mesh=_mesh,
    compiler_params=pltpu.CompilerParams(use_tc_tiling_on_sc=False),
    scratch_types=[
        pltpu.VMEM((NB, ECH), jnp.int32),
        pltpu.VMEM((ECH,), jnp.float32),
        pltpu.VMEM((RS,), jnp.float32),
        pltpu.VMEM_SHARED((NPAD,), jnp.float32),
    ],
)
def _deg_kernel(dst2d_hbm, out_hbm, didx, onesb, zb, degsh):
    c = lax.axis_index("c")
    s = lax.axis_index("s")
    w = s * NC + c

    @pl.loop(0, ECH // 16)
    def _(j):
        onesb[pl.ds(j * 16, 16)] = jnp.full((16,), 1.0, jnp.float32)

    @pl.loop(0, RS // 16)
    def _(j):
        zb[pl.ds(j * 16, 16)] = jnp.zeros((16,), jnp.float32)

    pltpu.sync_copy(zb, degsh.at[pl.ds(s * RS, RS)])
    plsc.subcore_barrier()

    cb = w * (EPAD // NW // ECH)

    @pl.loop(0, EPAD // NW // ECH, step=NB)
    def _(g0):
        pltpu.sync_copy(dst2d_hbm.at[pl.ds(cb + g0, NB), :], didx)
        for b in range(NB):
            pltpu.sync_copy(onesb, degsh.at[didx.at[b]], add=True)

    plsc.subcore_barrier()
    pltpu.sync_copy(degsh.at[pl.ds(s * RS, RS)], out_hbm.at[c, pl.ds(s * RS, RS)])


@functools.partial(
    pl.kernel,
    out_type=jax.ShapeDtypeStruct((NC, NPAD, DH), jnp.float32),
    mesh=_mesh,
    compiler_params=pltpu.CompilerParams(use_tc_tiling_on_sc=False),
    scratch_types=[
        pltpu.VMEM((SB, ECH), jnp.int32),
        pltpu.VMEM((SB, ECH), jnp.int32),
        pltpu.VMEM((SB, ECH), jnp.int32),
        pltpu.VMEM((SB, ECH), jnp.int32),
        pltpu.VMEM((SB * ECH, DH), jnp.float32),
        pltpu.VMEM((SB * ECH, DH), jnp.float32),
        pltpu.VMEM((128, DH), jnp.float32),
        pltpu.VMEM_SHARED((NPAD, DH), jnp.float32),
        pltpu.SemaphoreType.DMA,
        pltpu.SemaphoreType.DMA,
        pltpu.SemaphoreType.DMA,
        pltpu.SemaphoreType.DMA,
    ],
)
def _segsum_kernel(tab_hbm, src3d_hbm, dst2d_hbm, out_hbm,
                   sidxa, didxa, sidxb, didxb, rowsa, rowsb, zblk, ssh,
                   sga, sgb, ssa, ssb):
    c = lax.axis_index("c")
    s = lax.axis_index("s")

    @pl.loop(0, 128)
    def _(i):
        @pl.loop(0, DH // 16)
        def _(j):
            zblk[i, pl.ds(j * 16, 16)] = jnp.zeros((16,), jnp.float32)

    @pl.loop(0, RS // 128)
    def _(k):
        pltpu.sync_copy(zblk, ssh.at[pl.ds(s * RS + k * 128, 128), :])

    plsc.subcore_barrier()

    cb = s * CHT
    NG = CHT // SB

    def fire_g(grp, sidx, didx, rows, sem):
        row0 = cb + grp * SB
        pltpu.sync_copy(src3d_hbm.at[c, pl.ds(row0, SB), :], sidx)
        pltpu.sync_copy(dst2d_hbm.at[pl.ds(row0, SB), :], didx)
        for b in range(SB):
            pltpu.async_copy(tab_hbm.at[sidx.at[b]],
                             rows.at[pl.ds(b * ECH, ECH), :], sem)

    def wait_group(rows, sem):
        pltpu.make_async_copy(tab_hbm.at[pl.ds(0, SB * ECH), :], rows,
                              sem).wait()

    def fire_s(didx, rows, sem):
        for b in range(SB):
            pltpu.async_copy(rows.at[pl.ds(b * ECH, ECH), :],
                             ssh.at[didx.at[b]], sem, add=True)

    def wait_s(rows, sem):
        pltpu.make_async_copy(rows, ssh.at[pl.ds(0, SB * ECH), :], sem).wait()

    fire_g(0, sidxa, didxa, rowsa, sga)

    @pl.loop(0, NG // 2)
    def _(k):
        wait_group(rowsa, sga)
        fire_s(didxa, rowsa, ssa)

        @pl.when(k > 0)
        def _():
            wait_s(rowsb, ssb)

        fire_g(2 * k + 1, sidxb, didxb, rowsb, sgb)
        wait_group(rowsb, sgb)
        fire_s(didxb, rowsb, ssb)
        wait_s(rowsa, ssa)

        @pl.when(k < NG // 2 - 1)
        def _():
            fire_g(2 * k + 2, sidxa, didxa, rowsa, sga)

    wait_s(rowsb, ssb)

    plsc.subcore_barrier()
    pltpu.sync_copy(ssh.at[pl.ds(s * RS, RS), :],
                    out_hbm.at[c, pl.ds(s * RS, RS), :])


PCH = N // 16
GS = G // NS


@functools.partial(
    pl.kernel,
    out_type=(jax.ShapeDtypeStruct((NC, G, D), jnp.float32),
              jax.ShapeDtypeStruct((NC, G), jnp.float32)),
    mesh=_mesh,
    compiler_params=pltpu.CompilerParams(use_tc_tiling_on_sc=False),
    scratch_types=[
        pltpu.VMEM((16, DH), jnp.float32),
        pltpu.VMEM((16, DH), jnp.float32),
        pltpu.VMEM((16, DH), jnp.float32),
        pltpu.VMEM((16, DH), jnp.float32),
        pltpu.VMEM((16, D), jnp.float32),
        pltpu.VMEM((16,), jnp.float32),
        pltpu.VMEM((1, 16), jnp.int32),
        pltpu.VMEM((D,), jnp.float32),
        pltpu.VMEM((16,), jnp.float32),
        pltpu.VMEM((16, D), jnp.float32),
        pltpu.VMEM((16,), jnp.float32),
        pltpu.VMEM_SHARED((G, D), jnp.float32),
        pltpu.VMEM_SHARED((G,), jnp.float32),
    ],
)
def _pool_kernel(s2_hbm, h_hbm, dinv_hbm, batch_hbm, b2_hbm,
                 pout, cout,
                 bufsa, bufsb, bufha, bufhb, outb, dbuf, bidx, b2b, ones16,
                 z2d, z1d, psh, csh):
    c = lax.axis_index("c")
    s = lax.axis_index("s")
    w = s * NC + c

    ones16[...] = jnp.full((16,), 1.0, jnp.float32)
    z1d[...] = jnp.zeros((16,), jnp.float32)

    @pl.loop(0, 16)
    def _(i):
        @pl.loop(0, D // 16)
        def _(j):
            z2d[i, pl.ds(j * 16, 16)] = jnp.zeros((16,), jnp.float32)

    pltpu.sync_copy(b2_hbm, b2b)
    pltpu.sync_copy(z2d, psh.at[pl.ds(s * GS, GS), :])
    pltpu.sync_copy(z1d, csh.at[pl.ds(s * GS, GS)])
    plsc.subcore_barrier()

    @pl.loop(w, PCH, step=NW)
    def _(ch):
        r0 = ch * 16
        pltpu.sync_copy(s2_hbm.at[0, pl.ds(r0, 16), :], bufsa)
        pltpu.sync_copy(s2_hbm.at[1, pl.ds(r0, 16), :], bufsb)
        pltpu.sync_copy(h_hbm.at[0, pl.ds(r0, 16), :], bufha)
        pltpu.sync_copy(h_hbm.at[1, pl.ds(r0, 16), :], bufhb)
        pltpu.sync_copy(dinv_hbm.at[pl.ds(r0, 16)], dbuf)
        pltpu.sync_copy(batch_hbm.at[pl.ds(r0, 16)], bidx.at[0])
        dvec = dbuf[...]
        for r in range(16):
            dscal = dvec[r]
            for j in range(DH // 16):
                sl = pl.ds(j * 16, 16)
                v = (bufsa[r, sl] + bufha[r, sl]) * dscal + b2b[sl]
                outb[r, sl] = jnp.where(v >= 0.0, v, 0.01 * v)
                sh = pl.ds(DH + j * 16, 16)
                v2 = (bufsb[r, sl] + bufhb[r, sl]) * dscal + b2b[sh]
                outb[r, sh] = jnp.where(v2 >= 0.0, v2, 0.01 * v2)
        pltpu.sync_copy(outb, psh.at[bidx.at[0]], add=True)
        pltpu.sync_copy(ones16, csh.at[bidx.at[0]], add=True)

    plsc.subcore_barrier()
    pltpu.sync_copy(psh.at[pl.ds(s * GS, GS), :], pout.at[c, pl.ds(s * GS, GS), :])
    pltpu.sync_copy(csh.at[pl.ds(s * GS, GS)], cout.at[c, pl.ds(s * GS, GS)])


BR = 1024


def _k1_body(deg0_ref, deg1_ref, mask_ref, x_ref, w_ref,
             hcat_ref, dinv_ref):
    deg = deg0_ref[...] + deg1_ref[...] + mask_ref[...]
    dinv = jnp.where(deg > 0.0, lax.rsqrt(deg), 0.0)
    h = dinv * jnp.dot(x_ref[...], w_ref[...], preferred_element_type=jnp.float32)
    hcat_ref[0, :, :] = h[:, :DH]
    hcat_ref[1, :, :] = h[:, DH:]
    dinv_ref[...] = dinv


def _k3_body(s_ref, h_ref, dinv_ref, b_ref, w_ref, o_ref):
    dinv = dinv_ref[...]
    t = jnp.concatenate([s_ref[0, :, :] + h_ref[0, :, :],
                         s_ref[1, :, :] + h_ref[1, :, :]], axis=1)
    pre = dinv * t + b_ref[...]
    g = jnp.where(pre >= 0.0, pre, 0.01 * pre)
    h = dinv * jnp.dot(g, w_ref[...], preferred_element_type=jnp.float32)
    o_ref[0, :, :] = h[:, :DH]
    o_ref[1, :, :] = h[:, DH:]


def _k6_body(p0_ref, p1_ref, c0_ref, c1_ref, out_ref):
    cnt = jnp.maximum(c0_ref[...] + c1_ref[...], 1.0)
    out_ref[...] = (p0_ref[...] + p1_ref[...]) / cnt


_col_spec = pl.BlockSpec((BR, 1), lambda i: (i, 0))
_row_spec = pl.BlockSpec((BR, D), lambda i: (i, 0))
_stk_spec = pl.BlockSpec((2, BR, DH), lambda i: (0, i, 0))
_w_spec = pl.BlockSpec((D, D), lambda i: (0, 0))
_b_spec = pl.BlockSpec((1, D), lambda i: (0, 0))

_k1_call = pl.pallas_call(
    _k1_body,
    grid=(NPAD // BR,),
    in_specs=[_col_spec, _col_spec, _col_spec, _row_spec, _w_spec],
    out_specs=[_stk_spec, _col_spec],
    out_shape=[jax.ShapeDtypeStruct((2, NPAD, DH), jnp.float32),
               jax.ShapeDtypeStruct((NPAD, 1), jnp.float32)],
)

_k3_call = pl.pallas_call(
    _k3_body,
    grid=(NPAD // BR,),
    in_specs=[_stk_spec, _stk_spec, _col_spec, _b_spec, _w_spec],
    out_specs=_stk_spec,
    out_shape=jax.ShapeDtypeStruct((2, NPAD, DH), jnp.float32),
)

_k6_call = pl.pallas_call(
    _k6_body,
    in_specs=[pl.BlockSpec((G, D), lambda: (0, 0)),
              pl.BlockSpec((G, D), lambda: (0, 0)),
              pl.BlockSpec((G, 1), lambda: (0, 0)),
              pl.BlockSpec((G, 1), lambda: (0, 0))],
    out_specs=pl.BlockSpec((G, D), lambda: (0, 0)),
    out_shape=jax.ShapeDtypeStruct((G, D), jnp.float32),
)


def kernel(drug_x, edge_index, batch, W1, b1, W2, b2):
    f32 = jnp.float32
    i32 = jnp.int32

    x_pad = jnp.pad(drug_x.astype(f32), ((0, NPAD - N), (0, 0)))
    src = edge_index[0].astype(i32)
    dst = edge_index[1].astype(i32)
    pad_idx = jnp.full((EPAD - E,), N, dtype=i32)
    srcp = jnp.concatenate([src, pad_idx])
    dstp = jnp.concatenate([dst, pad_idx])
    batch32 = batch.astype(i32)
    mask_col = jnp.concatenate([jnp.ones((N, 1), f32), jnp.zeros((NPAD - N, 1), f32)])

    src2d = srcp.reshape(EPAD // ECH, ECH)
    dst2d = dstp.reshape(EPAD // ECH, ECH)
    src3d = jnp.stack([src2d, src2d + NPAD])

    degp = _deg_kernel(dst2d)
    h1, dinv = _k1_call(degp[0][:, None], degp[1][:, None], mask_col,
                        x_pad, W1.astype(f32))
    s1 = _segsum_kernel(h1.reshape(2 * NPAD, DH), src3d, dst2d)
    h2 = _k3_call(s1, h1, dinv, b1.astype(f32)[None, :], W2.astype(f32))
    s2 = _segsum_kernel(h2.reshape(2 * NPAD, DH), src3d, dst2d)
    poolp, cntp = _pool_kernel(s2, h2, dinv[:, 0], batch32,
                               b2.astype(f32))
    out = _k6_call(poolp[0], poolp[1], cntp[0][:, None], cntp[1][:, None])
    return out


# speedup vs baseline: 12.9769x; 1.0852x over previous
import functools

import jax
import jax.numpy as jnp
from jax import lax
from jax.experimental import pallas as pl
from jax.experimental.pallas import tpu as pltpu
from jax.experimental.pallas import tpu_sc as plsc

N = 10000
E = 320000
D = 128
DH = D // 2
G = 256

NC = 2
NS = 16
NW = NC * NS

NPAD = 10240
EPAD = 327680
ECH = 128
ET = EPAD // NS
CHT = ET // ECH
RS = NPAD // NS
NB = 4
SB = 4

_mesh = plsc.VectorSubcoreMesh(core_axis_name="c", subcore_axis_name="s")


@functools.partial(
    pl.kernel,
    out_type=jax.ShapeDtypeStruct((NC, NPAD), jnp.float32),
    mesh=_mesh,
    compiler_params=pltpu.CompilerParams(use_tc_tiling_on_sc=False),
    scratch_types=[
        pltpu.VMEM((NB, ECH), jnp.int32),
        pltpu.VMEM((NB, ECH), jnp.int32),
        pltpu.VMEM((ECH,), jnp.float32),
        pltpu.VMEM((RS,), jnp.float32),
        pltpu.VMEM_SHARED((NPAD,), jnp.float32),
        pltpu.SemaphoreType.DMA,
        pltpu.SemaphoreType.DMA,
    ],
)
def _deg_kernel(dst2d_hbm, out_hbm, didxa, didxb, onesb, zb, degsh,
                sma, smb):
    c = lax.axis_index("c")
    s = lax.axis_index("s")
    w = s * NC + c

    @pl.loop(0, ECH // 16)
    def _(j):
        onesb[pl.ds(j * 16, 16)] = jnp.full((16,), 1.0, jnp.float32)

    @pl.loop(0, RS // 16)
    def _(j):
        zb[pl.ds(j * 16, 16)] = jnp.zeros((16,), jnp.float32)

    pltpu.sync_copy(zb, degsh.at[pl.ds(s * RS, RS)])
    plsc.subcore_barrier()

    CHD = EPAD // NW // ECH
    cb = w * CHD

    def fire(grp, didx, sem):
        pltpu.sync_copy(dst2d_hbm.at[pl.ds(cb + grp * NB, NB), :], didx)
        for b in range(NB):
            pltpu.async_copy(onesb, degsh.at[didx.at[b]], sem, add=True)

    def wait(didx, sem):
        for b in range(NB):
            pltpu.make_async_copy(onesb, degsh.at[didx.at[b]], sem).wait()

    fire(0, didxa, sma)

    @pl.loop(0, CHD // NB // 2)
    def _(k):
        fire(2 * k + 1, didxb, smb)
        wait(didxa, sma)

        @pl.when(k < CHD // NB // 2 - 1)
        def _():
            fire(2 * k + 2, didxa, sma)

        wait(didxb, smb)

    plsc.subcore_barrier()
    pltpu.sync_copy(degsh.at[pl.ds(s * RS, RS)], out_hbm.at[c, pl.ds(s * RS, RS)])


@functools.partial(
    pl.kernel,
    out_type=jax.ShapeDtypeStruct((NC, NPAD, DH), jnp.float32),
    mesh=_mesh,
    compiler_params=pltpu.CompilerParams(use_tc_tiling_on_sc=False),
    scratch_types=[
        pltpu.VMEM((SB, ECH), jnp.int32),
        pltpu.VMEM((SB, ECH), jnp.int32),
        pltpu.VMEM((SB, ECH), jnp.int32),
        pltpu.VMEM((SB, ECH), jnp.int32),
        pltpu.VMEM((SB * ECH, DH), jnp.float32),
        pltpu.VMEM((SB * ECH, DH), jnp.float32),
        pltpu.VMEM((128, DH), jnp.float32),
        pltpu.VMEM_SHARED((NPAD, DH), jnp.float32),
        pltpu.SemaphoreType.DMA,
        pltpu.SemaphoreType.DMA,
        pltpu.SemaphoreType.DMA,
        pltpu.SemaphoreType.DMA,
    ],
)
def _segsum_kernel(tab_hbm, src3d_hbm, dst2d_hbm, out_hbm,
                   sidxa, didxa, sidxb, didxb, rowsa, rowsb, zblk, ssh,
                   sga, sgb, ssa, ssb):
    c = lax.axis_index("c")
    s = lax.axis_index("s")

    @pl.loop(0, 128)
    def _(i):
        @pl.loop(0, DH // 16)
        def _(j):
            zblk[i, pl.ds(j * 16, 16)] = jnp.zeros((16,), jnp.float32)

    @pl.loop(0, RS // 128)
    def _(k):
        pltpu.sync_copy(zblk, ssh.at[pl.ds(s * RS + k * 128, 128), :])

    plsc.subcore_barrier()

    cb = s * CHT
    NG = CHT // SB

    def fire_g(grp, sidx, didx, rows, sem):
        row0 = cb + grp * SB
        pltpu.sync_copy(src3d_hbm.at[c, pl.ds(row0, SB), :], sidx)
        pltpu.sync_copy(dst2d_hbm.at[pl.ds(row0, SB), :], didx)
        for b in range(SB):
            pltpu.async_copy(tab_hbm.at[sidx.at[b]],
                             rows.at[pl.ds(b * ECH, ECH), :], sem)

    def wait_group(rows, sem):
        pltpu.make_async_copy(tab_hbm.at[pl.ds(0, SB * ECH), :], rows,
                              sem).wait()

    def fire_s(didx, rows, sem):
        for b in range(SB):
            pltpu.async_copy(rows.at[pl.ds(b * ECH, ECH), :],
                             ssh.at[didx.at[b]], sem, add=True)

    def wait_s(rows, sem):
        pltpu.make_async_copy(rows, ssh.at[pl.ds(0, SB * ECH), :], sem).wait()

    fire_g(0, sidxa, didxa, rowsa, sga)

    @pl.loop(0, NG // 2)
    def _(k):
        wait_group(rowsa, sga)
        fire_s(didxa, rowsa, ssa)

        @pl.when(k > 0)
        def _():
            wait_s(rowsb, ssb)

        fire_g(2 * k + 1, sidxb, didxb, rowsb, sgb)
        wait_group(rowsb, sgb)
        fire_s(didxb, rowsb, ssb)
        wait_s(rowsa, ssa)

        @pl.when(k < NG // 2 - 1)
        def _():
            fire_g(2 * k + 2, sidxa, didxa, rowsa, sga)

    wait_s(rowsb, ssb)

    plsc.subcore_barrier()
    pltpu.sync_copy(ssh.at[pl.ds(s * RS, RS), :],
                    out_hbm.at[c, pl.ds(s * RS, RS), :])


GP = G + 16
GS = G // NS


@functools.partial(
    pl.kernel,
    out_type=(jax.ShapeDtypeStruct((NC, G, DH), jnp.float32),
              jax.ShapeDtypeStruct((NC, G), jnp.float32)),
    mesh=_mesh,
    compiler_params=pltpu.CompilerParams(use_tc_tiling_on_sc=False),
    scratch_types=[
        pltpu.VMEM((SB, ECH), jnp.int32),
        pltpu.VMEM((SB, ECH), jnp.int32),
        pltpu.VMEM((SB, ECH), jnp.int32),
        pltpu.VMEM((SB, ECH), jnp.int32),
        pltpu.VMEM((SB * ECH, DH), jnp.float32),
        pltpu.VMEM((SB * ECH, DH), jnp.float32),
        pltpu.VMEM((128, DH), jnp.float32),
        pltpu.VMEM((64, DH), jnp.float32),
        pltpu.VMEM((64, DH), jnp.float32),
        pltpu.VMEM((64, DH), jnp.float32),
        pltpu.VMEM((64,), jnp.float32),
        pltpu.VMEM((1, 64), jnp.int32),
        pltpu.VMEM((DH,), jnp.float32),
        pltpu.VMEM((64,), jnp.float32),
        pltpu.VMEM((GP,), jnp.float32),
        pltpu.VMEM_SHARED((NPAD, DH), jnp.float32),
        pltpu.VMEM_SHARED((GP, DH), jnp.float32),
        pltpu.VMEM_SHARED((GP,), jnp.float32),
        pltpu.SemaphoreType.DMA,
        pltpu.SemaphoreType.DMA,
        pltpu.SemaphoreType.DMA,
        pltpu.SemaphoreType.DMA,
    ],
)
def _segsum_pool_kernel(tab_hbm, src3d_hbm, dst2d_hbm, dinv_hbm, batchp_hbm,
                        b2s_hbm, pout, cout,
                        sidxa, didxa, sidxb, didxb, rowsa, rowsb, zblk,
                        sblk, ehbuf, outblk, dchunk, bidx, b2b, ones64, zc,
                        ssh, psh, csh, sga, sgb, ssa, ssb):
    c = lax.axis_index("c")
    s = lax.axis_index("s")

    @pl.loop(0, 128)
    def _(i):
        @pl.loop(0, DH // 16)
        def _(j):
            zblk[i, pl.ds(j * 16, 16)] = jnp.zeros((16,), jnp.float32)

    @pl.loop(0, 64 // 16)
    def _(j):
        ones64[pl.ds(j * 16, 16)] = jnp.full((16,), 1.0, jnp.float32)

    @pl.loop(0, GP // 16)
    def _(j):
        zc[pl.ds(j * 16, 16)] = jnp.zeros((16,), jnp.float32)

    @pl.loop(0, RS // 128)
    def _(k):
        pltpu.sync_copy(zblk, ssh.at[pl.ds(s * RS + k * 128, 128), :])

    pltpu.sync_copy(zblk.at[pl.ds(0, GP // NS), :],
                    psh.at[pl.ds(s * (GP // NS), GP // NS), :])

    @pl.when(s == 0)
    def _():
        pltpu.sync_copy(zc, csh)

    pltpu.sync_copy(b2s_hbm.at[c], b2b)
    plsc.subcore_barrier()

    cb = s * CHT
    NG = CHT // SB

    def fire_g(grp, sidx, didx, rows, sem):
        row0 = cb + grp * SB
        pltpu.sync_copy(src3d_hbm.at[c, pl.ds(row0, SB), :], sidx)
        pltpu.sync_copy(dst2d_hbm.at[pl.ds(row0, SB), :], didx)
        for b in range(SB):
            pltpu.async_copy(tab_hbm.at[sidx.at[b]],
                             rows.at[pl.ds(b * ECH, ECH), :], sem)

    def wait_group(rows, sem):
        pltpu.make_async_copy(tab_hbm.at[pl.ds(0, SB * ECH), :], rows,
                              sem).wait()

    def fire_s(didx, rows, sem):
        for b in range(SB):
            pltpu.async_copy(rows.at[pl.ds(b * ECH, ECH), :],
                             ssh.at[didx.at[b]], sem, add=True)

    def wait_s(rows, sem):
        pltpu.make_async_copy(rows, ssh.at[pl.ds(0, SB * ECH), :], sem).wait()

    fire_g(0, sidxa, didxa, rowsa, sga)

    @pl.loop(0, NG // 2)
    def _(k):
        wait_group(rowsa, sga)
        fire_s(didxa, rowsa, ssa)

        @pl.when(k > 0)
        def _():
            wait_s(rowsb, ssb)

        fire_g(2 * k + 1, sidxb, didxb, rowsb, sgb)
        wait_group(rowsb, sgb)
        fire_s(didxb, rowsb, ssb)
        wait_s(rowsa, ssa)

        @pl.when(k < NG // 2 - 1)
        def _():
            fire_g(2 * k + 2, sidxa, didxa, rowsa, sga)

    wait_s(rowsb, ssb)

    plsc.subcore_barrier()

    @pl.loop(0, RS // 64)
    def _(grp):
        r0 = s * RS + grp * 64
        pltpu.sync_copy(ssh.at[pl.ds(r0, 64), :], sblk)
        pltpu.sync_copy(tab_hbm.at[pl.ds(c * NPAD + r0, 64), :], ehbuf)
        pltpu.sync_copy(dinv_hbm.at[pl.ds(r0, 64)], dchunk)
        pltpu.sync_copy(batchp_hbm.at[pl.ds(r0, 64)], bidx.at[0])

        @pl.loop(0, 4)
        def _(rr):
            dvec = dchunk[pl.ds(rr * 16, 16)]
            for r in range(16):
                row = rr * 16 + r
                dscal = dvec[r]
                for j in range(DH // 16):
                    sl = pl.ds(j * 16, 16)
                    v = (sblk[row, sl] + ehbuf[row, sl]) * dscal + b2b[sl]
                    outblk[row, sl] = jnp.where(v >= 0.0, v, 0.01 * v)

        pltpu.sync_copy(outblk, psh.at[bidx.at[0]], add=True)

        @pl.when(c == 0)
        def _():
            pltpu.sync_copy(ones64, csh.at[bidx.at[0]], add=True)

    plsc.subcore_barrier()
    pltpu.sync_copy(psh.at[pl.ds(s * GS, GS), :],
                    pout.at[c, pl.ds(s * GS, GS), :])
    pltpu.sync_copy(csh.at[pl.ds(s * GS, GS)], cout.at[c, pl.ds(s * GS, GS)])


BR = 1024


def _k1a_body(x_ref, w_ref, h_ref):
    h_ref[...] = jnp.dot(x_ref[...], w_ref[...], preferred_element_type=jnp.float32)


def _k1b_body(deg0_ref, deg1_ref, mask_ref, h_ref, hcat_ref, dinv_ref):
    deg = deg0_ref[...] + deg1_ref[...] + mask_ref[...]
    dinv = jnp.where(deg > 0.0, lax.rsqrt(deg), 0.0)
    h = dinv * h_ref[...]
    hcat_ref[0, :, :] = h[:, :DH]
    hcat_ref[1, :, :] = h[:, DH:]
    dinv_ref[...] = dinv


def _k3_body(s_ref, h_ref, dinv_ref, b_ref, w_ref, o_ref):
    dinv = dinv_ref[...]
    t = jnp.concatenate([s_ref[0, :, :] + h_ref[0, :, :],
                         s_ref[1, :, :] + h_ref[1, :, :]], axis=1)
    pre = dinv * t + b_ref[...]
    g = jnp.where(pre >= 0.0, pre, 0.01 * pre)
    h = dinv * jnp.dot(g, w_ref[...], preferred_element_type=jnp.float32)
    o_ref[0, :, :] = h[:, :DH]
    o_ref[1, :, :] = h[:, DH:]


def _k6_body(p0_ref, p1_ref, c_ref, out_ref):
    cnt = jnp.maximum(c_ref[...], 1.0)
    out_ref[...] = jnp.concatenate([p0_ref[...], p1_ref[...]], axis=1) / cnt


_col_spec = pl.BlockSpec((BR, 1), lambda i: (i, 0))
_row_spec = pl.BlockSpec((BR, D), lambda i: (i, 0))
_stk_spec = pl.BlockSpec((2, BR, DH), lambda i: (0, i, 0))
_w_spec = pl.BlockSpec((D, D), lambda i: (0, 0))
_b_spec = pl.BlockSpec((1, D), lambda i: (0, 0))

_k1a_call = pl.pallas_call(
    _k1a_body,
    grid=(NPAD // BR,),
    in_specs=[_row_spec, _w_spec],
    out_specs=_row_spec,
    out_shape=jax.ShapeDtypeStruct((NPAD, D), jnp.float32),
)

_k1b_call = pl.pallas_call(
    _k1b_body,
    grid=(NPAD // BR,),
    in_specs=[_col_spec, _col_spec, _col_spec, _row_spec],
    out_specs=[_stk_spec, _col_spec],
    out_shape=[jax.ShapeDtypeStruct((2, NPAD, DH), jnp.float32),
               jax.ShapeDtypeStruct((NPAD, 1), jnp.float32)],
)

_k3_call = pl.pallas_call(
    _k3_body,
    grid=(NPAD // BR,),
    in_specs=[_stk_spec, _stk_spec, _col_spec, _b_spec, _w_spec],
    out_specs=_stk_spec,
    out_shape=jax.ShapeDtypeStruct((2, NPAD, DH), jnp.float32),
)

_k6_call = pl.pallas_call(
    _k6_body,
    in_specs=[pl.BlockSpec((G, DH), lambda: (0, 0)),
              pl.BlockSpec((G, DH), lambda: (0, 0)),
              pl.BlockSpec((G, 1), lambda: (0, 0))],
    out_specs=pl.BlockSpec((G, D), lambda: (0, 0)),
    out_shape=jax.ShapeDtypeStruct((G, D), jnp.float32),
)


def kernel(drug_x, edge_index, batch, W1, b1, W2, b2):
    f32 = jnp.float32
    i32 = jnp.int32

    x_pad = jnp.pad(drug_x.astype(f32), ((0, NPAD - N), (0, 0)))
    src = edge_index[0].astype(i32)
    dst = edge_index[1].astype(i32)
    pad_idx = jnp.full((EPAD - E,), N, dtype=i32)
    srcp = jnp.concatenate([src, pad_idx])
    dstp = jnp.concatenate([dst, pad_idx])
    batch32 = batch.astype(i32)
    mask_col = jnp.concatenate([jnp.ones((N, 1), f32), jnp.zeros((NPAD - N, 1), f32)])

    src2d = srcp.reshape(EPAD // ECH, ECH)
    dst2d = dstp.reshape(EPAD // ECH, ECH)
    src3d = jnp.stack([src2d, src2d + NPAD])

    batchp = jnp.concatenate([batch32, jnp.full((NPAD - N,), G, dtype=i32)])
    b2s = jnp.stack([b2.astype(f32)[:DH], b2.astype(f32)[DH:]])

    degp = _deg_kernel(dst2d)
    hraw = _k1a_call(x_pad, W1.astype(f32))
    h1, dinv = _k1b_call(degp[0][:, None], degp[1][:, None], mask_col,
                         hraw)
    s1 = _segsum_kernel(h1.reshape(2 * NPAD, DH), src3d, dst2d)
    h2 = _k3_call(s1, h1, dinv, b1.astype(f32)[None, :], W2.astype(f32))
    poolp, cntp = _segsum_pool_kernel(h2.reshape(2 * NPAD, DH), src3d, dst2d,
                                      dinv[:, 0], batchp, b2s)
    out = _k6_call(poolp[0], poolp[1], cntp[0][:, None])
    return out
